# Initial kernel scaffold; baseline (speedup 1.0000x reference)
import jax
import jax.numpy as jnp
from jax.experimental import pallas as pl


def kernel(x, shared_W, shared_b, gate_W, gate_b, gate_bias, expert_W, expert_b):
    raise NotImplementedError("write your pallas kernel here")


# speedup vs baseline: 1.2820x; 1.2820x over previous
import functools

import jax
import jax.numpy as jnp
from jax.experimental import pallas as pl
from jax.experimental.pallas import tpu as pltpu

T = 4096
D = 2048
E = 8
TM = 512


def _body(x_ref, w_ref, b_ref, gw_ref, gb_ref, out_ref, g_ref):
    j = pl.program_id(1)
    xf = x_ref[...]

    @pl.when(j == 0)
    def _():
        gs = jax.lax.dot_general(
            xf, gw_ref[...], (((1,), (1,)), ((), ())),
            preferred_element_type=jnp.float32,
        ) + gb_ref[...]
        lanes = jax.lax.broadcasted_iota(jnp.int32, (TM, E), 1)
        i0 = jnp.argmax(gs, axis=1)[:, None]
        masked = jnp.where(lanes == i0, -jnp.inf, gs)
        i1 = jnp.argmax(masked, axis=1)[:, None]
        m0 = jnp.max(gs, axis=1)[:, None]
        m1 = jnp.max(masked, axis=1)[:, None]
        e1 = jnp.exp(m1 - m0)
        w0 = 1.0 / (1.0 + e1)
        w1 = e1 / (1.0 + e1)
        g_ref[...] = jnp.where(lanes == i0, w0, 0.0) + jnp.where(lanes == i1, w1, 0.0)

    xb = xf.astype(jnp.bfloat16)
    mm = jax.lax.dot_general(
        xb, w_ref[0], (((1,), (1,)), ((), ())),
        preferred_element_type=jnp.float32,
    )

    @pl.when(j == 0)
    def _():
        out_ref[...] = mm + xf + b_ref[0]

    @pl.when(j > 0)
    def _():
        g = g_ref[...]
        lanes = jax.lax.broadcasted_iota(jnp.int32, (TM, E), 1)
        ge = jnp.sum(jnp.where(lanes == j - 1, g, 0.0), axis=1, keepdims=True)
        out_ref[...] += ge * (mm + b_ref[0])

    @pl.when(j == E)
    def _():
        out_ref[...] = jnp.maximum(out_ref[...], 0.0)


def kernel(x, shared_W, shared_b, gate_W, gate_b, gate_bias, expert_W, expert_b):
    W_all = jnp.concatenate([shared_W[None], expert_W], axis=0).astype(jnp.bfloat16)
    b_all = jnp.concatenate([shared_b[None], expert_b], axis=0).reshape(E + 1, 1, D)
    gb = (gate_b + gate_bias).reshape(1, E)

    grid = (T // TM, E + 1)
    out = pl.pallas_call(
        _body,
        grid=grid,
        in_specs=[
            pl.BlockSpec((TM, D), lambda i, j: (i, 0)),
            pl.BlockSpec((1, D, D), lambda i, j: (j, 0, 0)),
            pl.BlockSpec((1, 1, D), lambda i, j: (j, 0, 0)),
            pl.BlockSpec((E, D), lambda i, j: (0, 0)),
            pl.BlockSpec((1, E), lambda i, j: (0, 0)),
        ],
        out_specs=pl.BlockSpec((TM, D), lambda i, j: (i, 0)),
        out_shape=jax.ShapeDtypeStruct((T, D), jnp.float32),
        scratch_shapes=[pltpu.VMEM((TM, E), jnp.float32)],
        compiler_params=pltpu.CompilerParams(
            dimension_semantics=("parallel", "arbitrary"),
        ),
    )(x, W_all, b_all, gate_W, gb)
    return out
